# Initial kernel scaffold; baseline (speedup 1.0000x reference)
import jax
import jax.numpy as jnp
from jax.experimental import pallas as pl


def kernel(multivectors, multivectors_skip, scalars, scalars_skip, pos_source, pos_target, interp_source, interp_target, W_mv, W_s, b_s):
    raise NotImplementedError("write your pallas kernel here")


# speedup vs baseline: 25.3889x; 25.3889x over previous
import functools

import jax
import jax.numpy as jnp
from jax import lax
from jax.experimental import pallas as pl
from jax.experimental.pallas import tpu as pltpu
from jax.experimental.pallas import tpu_sc as plsc

N_SRC = 12500
N_TGT = 50000
E = 800000
C_IN = 4
S_IN = 16

ROW = 96
DEN_COL = 80
POS_COL = 80
T = 400
NB = N_TGT // T
K = 128
NW = 32
NB_PAD = 144


def _sc_body(packed_hbm, ptgt_hbm, esrc_hbm, etgt_hbm, bounds_hbm, acc_hbm,
             bounds_v, esrc_v, etgt_v, rows_v, ptgt_v, acc_v, sem):
    wid = lax.axis_index("s") * 2 + lax.axis_index("c")
    pltpu.sync_copy(bounds_hbm, bounds_v)
    nblk = (NB - wid + NW - 1) // NW

    def block_body(i, _):
        b = wid + i * NW
        bv = bounds_v[pl.ds(b, 16)]
        lo = bv[0]
        hi = bv[1]
        base_t = b * T

        def zrow(r, _):
            for c6 in range(ROW // 16):
                acc_v[r, pl.ds(c6 * 16, 16)] = jnp.zeros((16,), jnp.float32)
            return 0

        lax.fori_loop(0, T, zrow, 0)
        pltpu.sync_copy(ptgt_hbm.at[pl.ds(base_t, T)], ptgt_v)

        lo8 = lo & jnp.int32(-8)
        nch = (hi - lo8 + K - 1) // K

        def chunk_body(ch, _):
            start = pl.multiple_of(lo8 + ch * K, 8)
            pltpu.sync_copy(esrc_hbm.at[pl.ds(start, K)], esrc_v)
            pltpu.sync_copy(etgt_hbm.at[pl.ds(start, K)], etgt_v)
            pltpu.async_copy(packed_hbm.at[esrc_v], rows_v, sem).wait()

            def grp(j, _):
                e16 = j * 16 + lax.iota(jnp.int32, 16)
                ge = start + e16
                valid = (ge >= lo) & (ge < hi)
                tgt = etgt_v[pl.ds(j * 16, 16)]
                ltgt = tgt - base_t
                ltgt = jnp.clip(ltgt, 0, T - 1)

                def cvec(c):
                    return jnp.full((16,), c, jnp.int32)

                px = plsc.load_gather(rows_v, [e16, cvec(POS_COL)])
                py = plsc.load_gather(rows_v, [e16, cvec(POS_COL + 1)])
                pz = plsc.load_gather(rows_v, [e16, cvec(POS_COL + 2)])
                qx = plsc.load_gather(ptgt_v, [ltgt, cvec(0)])
                qy = plsc.load_gather(ptgt_v, [ltgt, cvec(1)])
                qz = plsc.load_gather(ptgt_v, [ltgt, cvec(2)])
                dx = px - qx
                dy = py - qy
                dz = pz - qz
                d2 = jnp.maximum(dx * dx + dy * dy + dz * dz,
                                 jnp.float32(1e-16))
                inv = 1.0 / d2
                inv = jnp.where(valid, inv, jnp.float32(0.0))
                plsc.addupdate_scatter(acc_v, [ltgt, cvec(DEN_COL)], inv)
                for c in range(80):
                    v = plsc.load_gather(rows_v, [e16, cvec(c)])
                    plsc.addupdate_scatter(acc_v, [ltgt, cvec(c)], v * inv)
                return 0

            lax.fori_loop(0, K // 16, grp, 0)
            return 0

        lax.fori_loop(0, nch, chunk_body, 0)
        pltpu.sync_copy(acc_v, acc_hbm.at[pl.ds(base_t, T)])
        return 0

    lax.fori_loop(0, nblk, block_body, 0)


_sc_interp = functools.partial(
    pl.kernel,
    _sc_body,
    out_type=jax.ShapeDtypeStruct((N_TGT, ROW), jnp.float32),
    mesh=plsc.VectorSubcoreMesh(core_axis_name="c", subcore_axis_name="s"),
    compiler_params=pltpu.CompilerParams(needs_layout_passes=False,
                                         use_tc_tiling_on_sc=False),
    scratch_types=[
        pltpu.VMEM((NB_PAD,), jnp.int32),
        pltpu.VMEM((K,), jnp.int32),
        pltpu.VMEM((K,), jnp.int32),
        pltpu.VMEM((K, ROW), jnp.float32),
        pltpu.VMEM((T, 4), jnp.float32),
        pltpu.VMEM((T, ROW), jnp.float32),
        pltpu.SemaphoreType.DMA,
    ],
)()


def _tc_body(acc_ref, mvskip_ref, sskip_ref, w1_ref, w2_ref, w3_ref,
             bias_ref, out_ref):
    acc = acc_ref[...]
    rec = 1.0 / acc[:, DEN_COL:DEN_COL + 1]
    xa = acc[:, :80] * rec
    out = jnp.dot(xa, w1_ref[...], preferred_element_type=jnp.float32)
    out += jnp.dot(mvskip_ref[...], w2_ref[...],
                   preferred_element_type=jnp.float32)
    out += jnp.dot(sskip_ref[...], w3_ref[...],
                   preferred_element_type=jnp.float32)
    out_ref[...] = out + bias_ref[...]


def _tc_mlp(acc, mvskip, sskip, w1, w2, w3, bias):
    R = 1000
    grid = N_TGT // R
    return pl.pallas_call(
        _tc_body,
        grid=(grid,),
        in_specs=[
            pl.BlockSpec((R, ROW), lambda i: (i, 0)),
            pl.BlockSpec((R, 64), lambda i: (i, 0)),
            pl.BlockSpec((R, S_IN), lambda i: (i, 0)),
            pl.BlockSpec((80, 80), lambda i: (0, 0)),
            pl.BlockSpec((64, 80), lambda i: (0, 0)),
            pl.BlockSpec((S_IN, 80), lambda i: (0, 0)),
            pl.BlockSpec((1, 80), lambda i: (0, 0)),
        ],
        out_specs=pl.BlockSpec((R, 80), lambda i: (i, 0)),
        out_shape=jax.ShapeDtypeStruct((N_TGT, 80), jnp.float32),
    )(acc, mvskip, sskip, w1, w2, w3, bias)


def kernel(multivectors, multivectors_skip, scalars, scalars_skip,
           pos_source, pos_target, interp_source, interp_target,
           W_mv, W_s, b_s):
    f32 = jnp.float32
    packed = jnp.concatenate(
        [multivectors.reshape(N_SRC, 64).astype(f32),
         scalars.astype(f32),
         pos_source.astype(f32),
         jnp.zeros((N_SRC, ROW - 83), f32)], axis=1)
    ptgt = jnp.concatenate(
        [pos_target.astype(f32), jnp.zeros((N_TGT, 1), f32)], axis=1)
    esrc = jnp.concatenate(
        [interp_source.astype(jnp.int32), jnp.zeros((K,), jnp.int32)])
    etgt = jnp.concatenate(
        [interp_target.astype(jnp.int32),
         jnp.full((K,), N_TGT - 1, jnp.int32)])
    starts = (jnp.arange(NB, dtype=jnp.int32) * T)
    bounds = jnp.searchsorted(interp_target.astype(jnp.int32), starts,
                              side="left").astype(jnp.int32)
    bounds = jnp.concatenate(
        [bounds, jnp.full((NB_PAD - NB,), E, jnp.int32)])

    acc = _sc_interp(packed, ptgt, esrc, etgt, bounds)

    eye = jnp.eye(16, dtype=f32)
    a_int = jnp.einsum("oc,de->cdoe", W_mv[:, :C_IN], eye).reshape(64, 64)
    a_skp = jnp.einsum("oc,de->cdoe", W_mv[:, C_IN:], eye).reshape(64, 64)
    w1 = jnp.zeros((80, 80), f32).at[:64, :64].set(a_int)
    w1 = w1.at[64:, 64:].set(W_s[:S_IN])
    w2 = jnp.concatenate([a_skp, jnp.zeros((64, 16), f32)], axis=1)
    w3 = jnp.concatenate([jnp.zeros((S_IN, 64), f32), W_s[S_IN:]], axis=1)
    bias = jnp.concatenate([jnp.zeros((64,), f32), b_s]).reshape(1, 80)

    out = _tc_mlp(acc, multivectors_skip.reshape(N_TGT, 64).astype(f32),
                  scalars_skip.astype(f32), w1, w2, w3, bias)
    mv_out = out[:, :64].reshape(N_TGT, C_IN, 16)
    s_out = out[:, 64:]
    return (mv_out, s_out)


# speedup vs baseline: 27.6644x; 1.0896x over previous
import functools

import jax
import jax.numpy as jnp
from jax import lax
from jax.experimental import pallas as pl
from jax.experimental.pallas import tpu as pltpu
from jax.experimental.pallas import tpu_sc as plsc

N_SRC = 12500
N_TGT = 50000
E = 800000
C_IN = 4
S_IN = 16

ROW = 96
DEN_COL = 80
POS_COL = 80
T = 400
NB = N_TGT // T
K = 128
NW = 32
CE = 2048
NB_PAD = 144


def _sc_body(packed_hbm, ptgt_hbm, esrc_hbm, etgt_hbm, bounds_hbm, acc_hbm,
             bounds_v, esrc_v, etgt_v, rows0_v, rows1_v, ptgt_v, acc_v,
             sem0, sem1):
    wid = lax.axis_index("s") * 2 + lax.axis_index("c")
    rows = (rows0_v, rows1_v)
    sems = (sem0, sem1)
    pltpu.sync_copy(bounds_hbm, bounds_v)
    nblk = (NB - wid + NW - 1) // NW

    def block_body(i, _):
        b = wid + i * NW
        bv = bounds_v[pl.ds(b, 16)]
        lo = bv[0]
        hi = bv[1]
        base_t = b * T

        def zrow(r, _):
            for c6 in range(ROW // 16):
                acc_v[r, pl.ds(c6 * 16, 16)] = jnp.zeros((16,), jnp.float32)
            return 0

        lax.fori_loop(0, T, zrow, 0)
        pltpu.sync_copy(ptgt_hbm.at[pl.ds(base_t, T)], ptgt_v)

        lo8 = lo & jnp.int32(-8)
        nslab = (hi - lo8 + CE - 1) // CE

        def slab_body(sl, _):
            sstart = pl.multiple_of(lo8 + sl * CE, 8)
            pltpu.sync_copy(esrc_hbm.at[pl.ds(sstart, CE)], esrc_v)
            pltpu.sync_copy(etgt_hbm.at[pl.ds(sstart, CE)], etgt_v)
            nck = (jnp.minimum(hi - sstart, CE) + K - 1) // K

            def issue(ch, par):
                pltpu.async_copy(
                    packed_hbm.at[esrc_v.at[pl.ds(ch * K, K)]],
                    rows[par], sems[par])

            @pl.when(nck > 0)
            def _():
                issue(0, 0)

            @pl.when(nck > 1)
            def _():
                issue(1, 1)

            def process(ch, par):
                def grp(j, _):
                    el16 = j * 16 + lax.iota(jnp.int32, 16)
                    e16 = ch * K + el16
                    ge = sstart + e16
                    valid = (ge >= lo) & (ge < hi)
                    tgt = etgt_v[pl.ds(ch * K + j * 16, 16)]
                    ltgt = tgt - base_t
                    ltgt = jnp.clip(ltgt, 0, T - 1)

                    def cvec(c):
                        return jnp.full((16,), c, jnp.int32)

                    px = plsc.load_gather(rows[par], [el16, cvec(POS_COL)])
                    py = plsc.load_gather(rows[par], [el16, cvec(POS_COL + 1)])
                    pz = plsc.load_gather(rows[par], [el16, cvec(POS_COL + 2)])
                    qx = plsc.load_gather(ptgt_v, [ltgt, cvec(0)])
                    qy = plsc.load_gather(ptgt_v, [ltgt, cvec(1)])
                    qz = plsc.load_gather(ptgt_v, [ltgt, cvec(2)])
                    dx = px - qx
                    dy = py - qy
                    dz = pz - qz
                    d2 = jnp.maximum(dx * dx + dy * dy + dz * dz,
                                     jnp.float32(1e-16))
                    inv = 1.0 / d2
                    inv = jnp.where(valid, inv, jnp.float32(0.0))
                    plsc.addupdate_scatter(acc_v, [ltgt, cvec(DEN_COL)], inv)
                    for c in range(80):
                        v = plsc.load_gather(rows[par], [el16, cvec(c)])
                        plsc.addupdate_scatter(acc_v, [ltgt, cvec(c)], v * inv)
                    return 0

                lax.fori_loop(0, K // 16, grp, 0)

            def pair_body(cc, _):
                for par in range(2):
                    ch = cc * 2 + par

                    @pl.when(ch < nck)
                    def _():
                        pltpu.make_async_copy(
                            packed_hbm.at[esrc_v.at[pl.ds(ch * K, K)]],
                            rows[par], sems[par]).wait()
                        process(ch, par)

                        @pl.when(ch + 2 < nck)
                        def _():
                            issue(ch + 2, par)
                return 0

            lax.fori_loop(0, (nck + 1) // 2, pair_body, 0)
            return 0

        lax.fori_loop(0, nslab, slab_body, 0)
        pltpu.sync_copy(acc_v, acc_hbm.at[pl.ds(base_t, T)])
        return 0

    lax.fori_loop(0, nblk, block_body, 0)


_sc_interp = functools.partial(
    pl.kernel,
    _sc_body,
    out_type=jax.ShapeDtypeStruct((N_TGT, ROW), jnp.float32),
    mesh=plsc.VectorSubcoreMesh(core_axis_name="c", subcore_axis_name="s"),
    compiler_params=pltpu.CompilerParams(needs_layout_passes=False,
                                         use_tc_tiling_on_sc=False),
    scratch_types=[
        pltpu.VMEM((NB_PAD,), jnp.int32),
        pltpu.VMEM((CE,), jnp.int32),
        pltpu.VMEM((CE,), jnp.int32),
        pltpu.VMEM((K, ROW), jnp.float32),
        pltpu.VMEM((K, ROW), jnp.float32),
        pltpu.VMEM((T, 4), jnp.float32),
        pltpu.VMEM((T, ROW), jnp.float32),
        pltpu.SemaphoreType.DMA,
        pltpu.SemaphoreType.DMA,
    ],
)()


def _tc_body(acc_ref, mvskip_ref, sskip_ref, w1_ref, w2_ref, w3_ref,
             bias_ref, out_ref):
    acc = acc_ref[...]
    rec = 1.0 / acc[:, DEN_COL:DEN_COL + 1]
    xa = acc[:, :80] * rec
    out = jnp.dot(xa, w1_ref[...], preferred_element_type=jnp.float32)
    out += jnp.dot(mvskip_ref[...], w2_ref[...],
                   preferred_element_type=jnp.float32)
    out += jnp.dot(sskip_ref[...], w3_ref[...],
                   preferred_element_type=jnp.float32)
    out_ref[...] = out + bias_ref[...]


def _tc_mlp(acc, mvskip, sskip, w1, w2, w3, bias):
    R = 1000
    grid = N_TGT // R
    return pl.pallas_call(
        _tc_body,
        grid=(grid,),
        in_specs=[
            pl.BlockSpec((R, ROW), lambda i: (i, 0)),
            pl.BlockSpec((R, 64), lambda i: (i, 0)),
            pl.BlockSpec((R, S_IN), lambda i: (i, 0)),
            pl.BlockSpec((80, 80), lambda i: (0, 0)),
            pl.BlockSpec((64, 80), lambda i: (0, 0)),
            pl.BlockSpec((S_IN, 80), lambda i: (0, 0)),
            pl.BlockSpec((1, 80), lambda i: (0, 0)),
        ],
        out_specs=pl.BlockSpec((R, 80), lambda i: (i, 0)),
        out_shape=jax.ShapeDtypeStruct((N_TGT, 80), jnp.float32),
    )(acc, mvskip, sskip, w1, w2, w3, bias)


def kernel(multivectors, multivectors_skip, scalars, scalars_skip,
           pos_source, pos_target, interp_source, interp_target,
           W_mv, W_s, b_s):
    f32 = jnp.float32
    packed = jnp.concatenate(
        [multivectors.reshape(N_SRC, 64).astype(f32),
         scalars.astype(f32),
         pos_source.astype(f32),
         jnp.zeros((N_SRC, ROW - 83), f32)], axis=1)
    ptgt = jnp.concatenate(
        [pos_target.astype(f32), jnp.zeros((N_TGT, 1), f32)], axis=1)
    esrc = jnp.concatenate(
        [interp_source.astype(jnp.int32), jnp.zeros((CE,), jnp.int32)])
    etgt = jnp.concatenate(
        [interp_target.astype(jnp.int32),
         jnp.full((CE,), N_TGT - 1, jnp.int32)])
    starts = (jnp.arange(NB, dtype=jnp.int32) * T)
    bounds = jnp.searchsorted(interp_target.astype(jnp.int32), starts,
                              side="left").astype(jnp.int32)
    bounds = jnp.concatenate(
        [bounds, jnp.full((NB_PAD - NB,), E, jnp.int32)])

    acc = _sc_interp(packed, ptgt, esrc, etgt, bounds)

    eye = jnp.eye(16, dtype=f32)
    a_int = jnp.einsum("oc,de->cdoe", W_mv[:, :C_IN], eye).reshape(64, 64)
    a_skp = jnp.einsum("oc,de->cdoe", W_mv[:, C_IN:], eye).reshape(64, 64)
    w1 = jnp.zeros((80, 80), f32).at[:64, :64].set(a_int)
    w1 = w1.at[64:, 64:].set(W_s[:S_IN])
    w2 = jnp.concatenate([a_skp, jnp.zeros((64, 16), f32)], axis=1)
    w3 = jnp.concatenate([jnp.zeros((S_IN, 64), f32), W_s[S_IN:]], axis=1)
    bias = jnp.concatenate([jnp.zeros((64,), f32), b_s]).reshape(1, 80)

    out = _tc_mlp(acc, multivectors_skip.reshape(N_TGT, 64).astype(f32),
                  scalars_skip.astype(f32), w1, w2, w3, bias)
    mv_out = out[:, :64].reshape(N_TGT, C_IN, 16)
    s_out = out[:, 64:]
    return (mv_out, s_out)


# speedup vs baseline: 178.5034x; 6.4525x over previous
import functools

import jax
import jax.numpy as jnp
from jax import lax
from jax.experimental import pallas as pl
from jax.experimental.pallas import tpu as pltpu
from jax.experimental.pallas import tpu_sc as plsc

N_SRC = 12500
N_TGT = 50000
E = 800000
C_IN = 4
S_IN = 16

ROW = 96
DEN_COL = 80
POS_COL = 80
T = 400
NB = N_TGT // T
K = 128
NW = 32
CE = 2048
NB_PAD = 144


def _sc_body(packed_hbm, ptgt_hbm, esrc_hbm, etgt_hbm, bounds_hbm, acc_hbm,
             bounds_v, esrc_v, etgt_v, rows0_v, rows1_v, ptgt_v, acc_v,
             cnt_v, off_v, inv_v, sem0, sem1):
    wid = lax.axis_index("s") * 2 + lax.axis_index("c")
    rows = (rows0_v, rows1_v)
    sems = (sem0, sem1)
    iota16 = lax.iota(jnp.int32, 16)
    onehot0 = (iota16 == 0).astype(jnp.float32)
    pltpu.sync_copy(bounds_hbm, bounds_v)
    nblk = (NB - wid + NW - 1) // NW

    def splat(x):
        return jnp.full((16,), x)

    def block_body(i, _):
        b = wid + i * NW
        bv = bounds_v[pl.ds(b, 16)]
        lo = bv[0]
        hi = bv[1]
        base_t = b * T

        def zrow(r, _):
            for c6 in range(ROW // 16):
                acc_v[r, pl.ds(c6 * 16, 16)] = jnp.zeros((16,), jnp.float32)
            return 0

        lax.fori_loop(0, T, zrow, 0)
        for g in range(T // 16):
            cnt_v[pl.ds(g * 16, 16)] = jnp.zeros((16,), jnp.int32)
        pltpu.sync_copy(ptgt_hbm.at[pl.ds(base_t, T)], ptgt_v)

        lo8 = lo & jnp.int32(-8)
        nslab = (hi - lo8 + CE - 1) // CE

        def hist_slab(sl, _):
            sstart = pl.multiple_of(lo8 + sl * CE, 8)
            pltpu.sync_copy(etgt_hbm.at[pl.ds(sstart, CE)], etgt_v)
            ngrp = (jnp.minimum(hi - sstart, CE) + 15) // 16

            def hgrp(j, _):
                ge = sstart + j * 16 + iota16
                valid = (ge >= lo) & (ge < hi)
                tgt = etgt_v[pl.ds(j * 16, 16)]
                ltgt = jnp.clip(tgt - base_t, 0, T - 1)
                ones = jnp.where(valid, jnp.int32(1), jnp.int32(0))
                plsc.addupdate_scatter(cnt_v, [ltgt], ones)
                return 0

            lax.fori_loop(0, ngrp, hgrp, 0)
            return 0

        lax.fori_loop(0, nslab, hist_slab, 0)

        def scan_grp(g, carry):
            cs = plsc.cumsum(cnt_v[pl.ds(g * 16, 16)]) + splat(carry)
            off_v[pl.ds(g * 16, 16)] = cs + splat(lo)
            return cs[15]

        lax.fori_loop(0, T // 16, scan_grp, jnp.int32(0))

        def slab_body(sl, _):
            sstart = pl.multiple_of(lo8 + sl * CE, 8)
            pltpu.sync_copy(esrc_hbm.at[pl.ds(sstart, CE)], esrc_v)
            pltpu.sync_copy(etgt_hbm.at[pl.ds(sstart, CE)], etgt_v)
            nck = (jnp.minimum(hi - sstart, CE) + K - 1) // K

            def issue(ch, par):
                pltpu.async_copy(
                    packed_hbm.at[esrc_v.at[pl.ds(ch * K, K)]],
                    rows[par], sems[par])

            @pl.when(nck > 0)
            def _():
                issue(0, 0)

            @pl.when(nck > 1)
            def _():
                issue(1, 1)

            def process(ch, par):
                gs = sstart + ch * K
                gs_v = jnp.maximum(gs, lo)
                ge_v = jnp.minimum(gs + K, hi)

                def grp(j, _):
                    el16 = j * 16 + iota16
                    ge = gs + el16
                    valid = (ge >= lo) & (ge < hi)
                    tgt = etgt_v[pl.ds(ch * K + j * 16, 16)]
                    ltgt = jnp.clip(tgt - base_t, 0, T - 1)
                    px = plsc.load_gather(rows[par], [el16, splat(POS_COL)])
                    py = plsc.load_gather(rows[par],
                                          [el16, splat(POS_COL + 1)])
                    pz = plsc.load_gather(rows[par],
                                          [el16, splat(POS_COL + 2)])
                    qx = plsc.load_gather(ptgt_v, [ltgt, splat(0)])
                    qy = plsc.load_gather(ptgt_v, [ltgt, splat(1)])
                    qz = plsc.load_gather(ptgt_v, [ltgt, splat(2)])
                    dx = px - qx
                    dy = py - qy
                    dz = pz - qz
                    d2 = jnp.maximum(dx * dx + dy * dy + dz * dz,
                                     jnp.float32(1e-16))
                    inv = jnp.where(valid, 1.0 / d2, jnp.float32(0.0))
                    inv_v[pl.ds(j * 16, 16)] = inv
                    return 0

                lax.fori_loop(0, K // 16, grp, 0)

                t_first = plsc.load_gather(
                    etgt_v, [splat(gs_v - sstart)])[0]
                t_last = plsc.load_gather(
                    etgt_v, [splat(ge_v - 1 - sstart)])[0]

                def tgt_body(t, _):
                    tl = t - base_t
                    end_t = plsc.load_gather(off_v, [splat(tl)])[0]
                    prev = plsc.load_gather(
                        off_v, [splat(jnp.maximum(tl - 1, 0))])[0]
                    start_t = jnp.where(tl == 0, lo, prev)
                    e0 = jnp.maximum(start_t, gs_v)
                    e1 = jnp.minimum(end_t, ge_v)
                    el0 = e0 - gs

                    def edge_body(k, carry):
                        el = el0 + k
                        iv = plsc.load_gather(inv_v, [splat(el)])
                        a = tuple(
                            carry[c6] + rows[par][el, pl.ds(c6 * 16, 16)] * iv
                            for c6 in range(5))
                        return a + (carry[5] + iv,)

                    zero = jnp.zeros((16,), jnp.float32)
                    a = lax.fori_loop(0, jnp.maximum(e1 - e0, 0), edge_body,
                                      (zero,) * 6)
                    for c6 in range(5):
                        plsc.addupdate(acc_v.at[tl, pl.ds(c6 * 16, 16)],
                                       a[c6])
                    plsc.addupdate(acc_v.at[tl, pl.ds(DEN_COL, 16)],
                                   a[5] * onehot0)
                    return 0

                lax.fori_loop(t_first, t_last + 1, tgt_body, 0)

            def pair_body(cc, _):
                for par in range(2):
                    ch = cc * 2 + par

                    @pl.when(ch < nck)
                    def _():
                        pltpu.make_async_copy(
                            packed_hbm.at[esrc_v.at[pl.ds(ch * K, K)]],
                            rows[par], sems[par]).wait()

                        @pl.when(jnp.maximum(sstart + ch * K, lo)
                                 < jnp.minimum(sstart + ch * K + K, hi))
                        def _():
                            process(ch, par)

                        @pl.when(ch + 2 < nck)
                        def _():
                            issue(ch + 2, par)
                return 0

            lax.fori_loop(0, (nck + 1) // 2, pair_body, 0)
            return 0

        lax.fori_loop(0, nslab, slab_body, 0)
        pltpu.sync_copy(acc_v, acc_hbm.at[pl.ds(base_t, T)])
        return 0

    lax.fori_loop(0, nblk, block_body, 0)


_sc_interp = functools.partial(
    pl.kernel,
    _sc_body,
    out_type=jax.ShapeDtypeStruct((N_TGT, ROW), jnp.float32),
    mesh=plsc.VectorSubcoreMesh(core_axis_name="c", subcore_axis_name="s"),
    compiler_params=pltpu.CompilerParams(needs_layout_passes=False,
                                         use_tc_tiling_on_sc=False),
    scratch_types=[
        pltpu.VMEM((NB_PAD,), jnp.int32),
        pltpu.VMEM((CE,), jnp.int32),
        pltpu.VMEM((CE,), jnp.int32),
        pltpu.VMEM((K, ROW), jnp.float32),
        pltpu.VMEM((K, ROW), jnp.float32),
        pltpu.VMEM((T, 4), jnp.float32),
        pltpu.VMEM((T, ROW), jnp.float32),
        pltpu.VMEM((T,), jnp.int32),
        pltpu.VMEM((T,), jnp.int32),
        pltpu.VMEM((K,), jnp.float32),
        pltpu.SemaphoreType.DMA,
        pltpu.SemaphoreType.DMA,
    ],
)()


def _tc_body(acc_ref, mvskip_ref, sskip_ref, w1_ref, w2_ref, w3_ref,
             bias_ref, out_ref):
    acc = acc_ref[...]
    rec = 1.0 / acc[:, DEN_COL:DEN_COL + 1]
    xa = acc[:, :80] * rec
    out = jnp.dot(xa, w1_ref[...], preferred_element_type=jnp.float32)
    out += jnp.dot(mvskip_ref[...], w2_ref[...],
                   preferred_element_type=jnp.float32)
    out += jnp.dot(sskip_ref[...], w3_ref[...],
                   preferred_element_type=jnp.float32)
    out_ref[...] = out + bias_ref[...]


def _tc_mlp(acc, mvskip, sskip, w1, w2, w3, bias):
    R = 1000
    grid = N_TGT // R
    return pl.pallas_call(
        _tc_body,
        grid=(grid,),
        in_specs=[
            pl.BlockSpec((R, ROW), lambda i: (i, 0)),
            pl.BlockSpec((R, 64), lambda i: (i, 0)),
            pl.BlockSpec((R, S_IN), lambda i: (i, 0)),
            pl.BlockSpec((80, 80), lambda i: (0, 0)),
            pl.BlockSpec((64, 80), lambda i: (0, 0)),
            pl.BlockSpec((S_IN, 80), lambda i: (0, 0)),
            pl.BlockSpec((1, 80), lambda i: (0, 0)),
        ],
        out_specs=pl.BlockSpec((R, 80), lambda i: (i, 0)),
        out_shape=jax.ShapeDtypeStruct((N_TGT, 80), jnp.float32),
    )(acc, mvskip, sskip, w1, w2, w3, bias)


def kernel(multivectors, multivectors_skip, scalars, scalars_skip,
           pos_source, pos_target, interp_source, interp_target,
           W_mv, W_s, b_s):
    f32 = jnp.float32
    packed = jnp.concatenate(
        [multivectors.reshape(N_SRC, 64).astype(f32),
         scalars.astype(f32),
         pos_source.astype(f32),
         jnp.zeros((N_SRC, ROW - 83), f32)], axis=1)
    ptgt = jnp.concatenate(
        [pos_target.astype(f32), jnp.zeros((N_TGT, 1), f32)], axis=1)
    esrc = jnp.concatenate(
        [interp_source.astype(jnp.int32), jnp.zeros((CE,), jnp.int32)])
    etgt = jnp.concatenate(
        [interp_target.astype(jnp.int32),
         jnp.full((CE,), N_TGT - 1, jnp.int32)])
    starts = (jnp.arange(NB, dtype=jnp.int32) * T)
    bounds = jnp.searchsorted(interp_target.astype(jnp.int32), starts,
                              side="left").astype(jnp.int32)
    bounds = jnp.concatenate(
        [bounds, jnp.full((NB_PAD - NB,), E, jnp.int32)])

    acc = _sc_interp(packed, ptgt, esrc, etgt, bounds)

    eye = jnp.eye(16, dtype=f32)
    a_int = jnp.einsum("oc,de->cdoe", W_mv[:, :C_IN], eye).reshape(64, 64)
    a_skp = jnp.einsum("oc,de->cdoe", W_mv[:, C_IN:], eye).reshape(64, 64)
    w1 = jnp.zeros((80, 80), f32).at[:64, :64].set(a_int)
    w1 = w1.at[64:, 64:].set(W_s[:S_IN])
    w2 = jnp.concatenate([a_skp, jnp.zeros((64, 16), f32)], axis=1)
    w3 = jnp.concatenate([jnp.zeros((S_IN, 64), f32), W_s[S_IN:]], axis=1)
    bias = jnp.concatenate([jnp.zeros((64,), f32), b_s]).reshape(1, 80)

    out = _tc_mlp(acc, multivectors_skip.reshape(N_TGT, 64).astype(f32),
                  scalars_skip.astype(f32), w1, w2, w3, bias)
    mv_out = out[:, :64].reshape(N_TGT, C_IN, 16)
    s_out = out[:, 64:]
    return (mv_out, s_out)
